# baseline (speedup 1.0000x reference)
import functools

import jax
import jax.numpy as jnp
import numpy as _np
from jax import lax
from jax.experimental import pallas as pl
from jax.experimental.pallas import tpu as pltpu
from jax.experimental.pallas import tpu_sc as plsc

_N = 10000
_NP = 10240
_E = 320000
_D = 128
_CH = 128
_NC, _NS = 2, 16
_EP = 327680
_NCH = _EP // _CH
_RPT = _NCH // (_NC * _NS)
_RT = _NP // _NS
_DEGW = 128
_RB = 2048

_mesh = plsc.VectorSubcoreMesh(
    core_axis_name="c", subcore_axis_name="s", num_cores=_NC, num_subcores=_NS
)


@functools.partial(
    pl.kernel,
    out_type=jax.ShapeDtypeStruct((_NC, _NP, _DEGW), jnp.float32),
    mesh=_mesh,
    scratch_types=[
        pltpu.VMEM((_RPT, _CH), jnp.int32),
        pltpu.VMEM((_CH, _DEGW), jnp.float32),
        pltpu.VMEM((_CH, _DEGW), jnp.float32),
        pltpu.VMEM_SHARED((_NP, _DEGW), jnp.float32),
        pltpu.SemaphoreType.DMA,
    ],
)
def _deg_kernel(e3_hbm, degp_hbm, idx_v, ones_v, zero_v, deg_sh, sem):
    c = lax.axis_index("c")
    s = lax.axis_index("s")
    rowbase = (c * _NS + s) * _RPT
    tb = s * _RT

    def fill(i, carry):
        for g in range(_DEGW // 16):
            ones_v[i, pl.ds(g * 16, 16)] = jnp.ones((16,), jnp.float32)
            zero_v[i, pl.ds(g * 16, 16)] = jnp.zeros((16,), jnp.float32)
        return carry

    lax.fori_loop(0, _CH, fill, 0)

    def zero_acc(k, carry):
        pltpu.sync_copy(zero_v, deg_sh.at[pl.ds(tb + k * _CH, _CH)])
        return carry

    lax.fori_loop(0, _RT // _CH, zero_acc, 0)
    plsc.subcore_barrier()

    pltpu.sync_copy(e3_hbm.at[1, pl.ds(rowbase, _RPT)], idx_v)

    def chunk(j, carry):
        pltpu.sync_copy(ones_v, deg_sh.at[idx_v.at[j]], add=True)
        return carry

    lax.fori_loop(0, _RPT, chunk, 0)
    plsc.subcore_barrier()
    pltpu.sync_copy(deg_sh.at[pl.ds(tb, _RT)], degp_hbm.at[c, pl.ds(tb, _RT)])


_G = 8


@functools.partial(
    pl.kernel,
    out_type=jax.ShapeDtypeStruct((_NC, _NP, _D), jnp.float32),
    mesh=_mesh,
    scratch_types=[
        pltpu.VMEM((_G, _CH), jnp.int32),
        pltpu.VMEM((_G, _CH), jnp.int32),
        pltpu.VMEM((_CH, _D), jnp.float32),
        pltpu.VMEM((_CH, _D), jnp.float32),
        pltpu.VMEM_SHARED((_NP, _D), jnp.float32),
        pltpu.SemaphoreType.DMA,
        pltpu.SemaphoreType.DMA,
    ],
)
def _scat_kernel(hn_hbm, e3_hbm, accp_hbm,
                 idxs_v, idxd_v, rows0, rows1, acc_sh, gsem, ssem):
    c = lax.axis_index("c")
    s = lax.axis_index("s")
    rowbase = (c * _NS + s) * _RPT
    tb = s * _RT

    def fill(i, carry):
        for k in range(_D // 16):
            rows0[i, pl.ds(k * 16, 16)] = jnp.zeros((16,), jnp.float32)
        return carry

    lax.fori_loop(0, _CH, fill, 0)

    def zero_acc(k, carry):
        pltpu.sync_copy(rows0, acc_sh.at[pl.ds(tb + k * _CH, _CH)])
        return carry

    lax.fori_loop(0, _RT // _CH, zero_acc, 0)
    plsc.subcore_barrier()

    def group(g, carry):
        rb = rowbase + g * _G
        pltpu.sync_copy(e3_hbm.at[0, pl.ds(rb, _G)], idxs_v)
        pltpu.sync_copy(e3_hbm.at[1, pl.ds(rb, _G)], idxd_v)
        pltpu.async_copy(hn_hbm.at[idxs_v.at[0]], rows0, gsem)
        for k in range(_G):
            buf, nbuf = (rows0, rows1) if k % 2 == 0 else (rows1, rows0)
            pltpu.make_async_copy(hn_hbm.at[idxs_v.at[k]], buf, gsem).wait()
            pltpu.async_copy(buf, acc_sh.at[idxd_v.at[k]], ssem, add=True)
            if k >= 1:
                pltpu.make_async_copy(nbuf, acc_sh.at[idxd_v.at[k - 1]], ssem).wait()
            if k + 1 < _G:
                pltpu.async_copy(hn_hbm.at[idxs_v.at[k + 1]], nbuf, gsem)
        pltpu.make_async_copy(rows1, acc_sh.at[idxd_v.at[_G - 1]], ssem).wait()
        return carry

    lax.fori_loop(0, _RPT // _G, group, 0)
    plsc.subcore_barrier()
    pltpu.sync_copy(acc_sh.at[pl.ds(tb, _RT)], accp_hbm.at[c, pl.ds(tb, _RT)])


def _tc1_body(x_ref, w_ref, degp_ref, hn_ref, dinv_ref):
    dp = degp_ref[...]
    deg = dp[0, :, 0:1] + dp[1, :, 0:1] + 1.0
    dinv = lax.rsqrt(deg)
    hn = jnp.dot(x_ref[...], w_ref[...], preferred_element_type=jnp.float32)
    hn_ref[...] = hn * dinv
    dinv_ref[...] = dinv


def _tc2_body(accp_ref, hn1_ref, dinv_ref, w_ref, b_ref, hn2_ref):
    ap = accp_ref[...]
    dinv = dinv_ref[...]
    h1 = jnp.maximum((ap[0] + ap[1] + hn1_ref[...]) * dinv + b_ref[...], 0.0)
    hn2_ref[...] = jnp.dot(h1, w_ref[...], preferred_element_type=jnp.float32) * dinv


def _tc3_body(accp_ref, hn2_ref, dinv_ref, b_ref, out_ref):
    ap = accp_ref[...]
    out_ref[...] = (ap[0] + ap[1] + hn2_ref[...]) * dinv_ref[...] + b_ref[...]


_PAD = _np.broadcast_to(
    _N + (_np.arange(_EP - _E, dtype=_np.int32) % (_NP - _N)), (2, _EP - _E))


def kernel(x, edge_index, W1, b1, W2, b2):
    ei = edge_index.astype(jnp.int32)
    e3 = jnp.concatenate([ei, jnp.asarray(_PAD)], axis=1).reshape(2, _NCH, _CH)

    degp = _deg_kernel(e3)

    nb = _NP // _RB
    hn1, dinv = pl.pallas_call(
        _tc1_body,
        grid=(nb,),
        in_specs=[
            pl.BlockSpec((_RB, _D), lambda i: (i, 0)),
            pl.BlockSpec((_D, _D), lambda i: (0, 0)),
            pl.BlockSpec((_NC, _RB, _DEGW), lambda i: (0, i, 0)),
        ],
        out_specs=[
            pl.BlockSpec((_RB, _D), lambda i: (i, 0)),
            pl.BlockSpec((_RB, 1), lambda i: (i, 0)),
        ],
        out_shape=[
            jax.ShapeDtypeStruct((_NP, _D), jnp.float32),
            jax.ShapeDtypeStruct((_NP, 1), jnp.float32),
        ],
    )(x, W1, degp)

    accp1 = _scat_kernel(hn1, e3)

    hn2 = pl.pallas_call(
        _tc2_body,
        grid=(nb,),
        in_specs=[
            pl.BlockSpec((_NC, _RB, _D), lambda i: (0, i, 0)),
            pl.BlockSpec((_RB, _D), lambda i: (i, 0)),
            pl.BlockSpec((_RB, 1), lambda i: (i, 0)),
            pl.BlockSpec((_D, _D), lambda i: (0, 0)),
            pl.BlockSpec((1, _D), lambda i: (0, 0)),
        ],
        out_specs=pl.BlockSpec((_RB, _D), lambda i: (i, 0)),
        out_shape=jax.ShapeDtypeStruct((_NP, _D), jnp.float32),
    )(accp1, hn1, dinv, W2, b1.reshape(1, _D))

    accp2 = _scat_kernel(hn2, e3)

    nb3 = _N // 2000
    out = pl.pallas_call(
        _tc3_body,
        grid=(nb3,),
        in_specs=[
            pl.BlockSpec((_NC, 2000, _D), lambda i: (0, i, 0)),
            pl.BlockSpec((2000, _D), lambda i: (i, 0)),
            pl.BlockSpec((2000, 1), lambda i: (i, 0)),
            pl.BlockSpec((1, _D), lambda i: (0, 0)),
        ],
        out_specs=pl.BlockSpec((2000, _D), lambda i: (i, 0)),
        out_shape=jax.ShapeDtypeStruct((_N, _D), jnp.float32),
    )(accp2, hn2, dinv, b2.reshape(1, _D))
    return out
